# baseline (speedup 1.0000x reference)
import functools
import jax
import jax.numpy as jnp
from jax import lax
from jax.experimental import pallas as pl
from jax.experimental.pallas import tpu as pltpu
from jax.experimental.pallas import tpu_sc as plsc

HEADS = 2
HIDDEN = 128
OUT_C = 64
D_IN = 128

NC, NS, L = 2, 16, 16
NW = NC * NS

F32 = jnp.float32
I32 = jnp.int32


ROUND_E = 256
CHUNK_A = 400
CPT = 2


def _make_edge_kernel(n_src, keep_n, E, width):
    E_t = E // NW
    CAP = ((E_t + ROUND_E - 1) // ROUND_E) * ROUND_E
    NRMAX = CAP // ROUND_E
    DEN = ((keep_n + 15) // 16) * 16
    ACC_R = ((keep_n + 15) // 16) * 16
    n_chunks = width // L
    n_groups = n_chunks // CPT
    n_splits = NS // n_groups
    G = NC * n_splits
    half_chunks = n_chunks // 2
    WT = CPT * L

    stage_tab = n_src * width * 4 <= 1_000_000
    rows_per = n_src // n_splits

    mesh = plsc.VectorSubcoreMesh(core_axis_name="c", subcore_axis_name="s",
                                  num_cores=NC, num_subcores=NS)

    def body(esrc, edst, ht_hbm, tabs_src, tabs_dst,
             acc_out, den_out,
             as0_v, as1_v, ad0_v, ad1_v, c0_v, c1_v,
             den0_v, den1_v, es_v, ed_v,
             cs_v, ce0_v, ce1_v,
             pk_r, sidx_r, ex_r, rows_r,
             pk_b, sidx_b, ex_b, rows_b, acc_v0, acc_v1,
             src_sp, exh_sp, counts_sp,
             cnt1_sm, cnts_sm, sem, sem_b, *maybe_ht_sp):
        accs_v = (acc_v0, acc_v1)
        ht_sp = maybe_ht_sp[0] if stage_tab else None
        cid = lax.axis_index("c")
        sid = lax.axis_index("s")
        wid = sid * NC + cid
        group = sid % n_groups
        split = sid // n_groups
        hsel = (group * CPT) // half_chunks

        pltpu.sync_copy(tabs_src.at[0], as0_v)
        pltpu.sync_copy(tabs_src.at[1], as1_v)
        pltpu.sync_copy(tabs_dst.at[0], ad0_v)
        pltpu.sync_copy(tabs_dst.at[1], ad1_v)
        pltpu.sync_copy(tabs_dst.at[2], c0_v)
        pltpu.sync_copy(tabs_dst.at[3], c1_v)
        if stage_tab:
            gi = sid % n_groups
            ro = (sid // n_groups) * rows_per
            pltpu.sync_copy(ht_hbm.at[gi, pl.ds(ro, rows_per)],
                            ht_sp.at[gi, pl.ds(ro, rows_per)])

        zf = jnp.zeros((L,), F32)
        zi = jnp.zeros((L,), I32)
        lane = lax.iota(I32, L)

        def zero_body(i, _):
            sl = pl.ds(i * L, L)
            cs_v[sl] = zi
            ce0_v[sl] = zf
            ce1_v[sl] = zf
            return 0
        lax.fori_loop(0, CAP // L, zero_body, 0)

        def zero_den(i, _):
            sl = pl.ds(i * L, L)
            den0_v[sl] = zf
            den1_v[sl] = zf
            return 0
        lax.fori_loop(0, DEN // L, zero_den, 0)

        def zero_acc(i, _):
            iv = jnp.broadcast_to(i, (L,))
            for j in range(CPT):
                plsc.store_scatter(accs_v[j], [iv, lane], zf)
            return 0
        lax.fori_loop(0, ACC_R, zero_acc, 0)

        def chunk_a(co, offset0):
            pltpu.sync_copy(esrc.at[pl.ds(wid * E_t + co * CHUNK_A, CHUNK_A)],
                            es_v)
            pltpu.sync_copy(edst.at[pl.ds(wid * E_t + co * CHUNK_A, CHUNK_A)],
                            ed_v)
            return lax.fori_loop(0, CHUNK_A // L, phase_a, offset0)

        def phase_a(i, offset):
            sl = pl.ds(i * L, L)
            src16 = es_v[sl]
            dst16 = ed_v[sl]
            keep = dst16 < keep_n
            dstc = jnp.where(keep, dst16, 0)
            a_s0 = plsc.load_gather(as0_v, [src16])
            a_s1 = plsc.load_gather(as1_v, [src16])
            a_d0 = plsc.load_gather(ad0_v, [dstc])
            a_d1 = plsc.load_gather(ad1_v, [dstc])
            cc0 = plsc.load_gather(c0_v, [dstc])
            cc1 = plsc.load_gather(c1_v, [dstc])
            z0 = a_s0 + a_d0
            z1 = a_s1 + a_d1
            al0 = jnp.where(z0 >= 0, z0, 0.2 * z0)
            al1 = jnp.where(z1 >= 0, z1, 0.2 * z1)
            ex0 = jnp.exp(al0 - cc0)
            ex1 = jnp.exp(al1 - cc1)
            plsc.addupdate_scatter(den0_v, [dstc], ex0, mask=keep)
            plsc.addupdate_scatter(den1_v, [dstc], ex1, mask=keep)
            ki = keep.astype(I32)
            pos = offset + plsc.cumsum(ki) - ki
            packed = lax.shift_left(src16, 10) | dstc
            plsc.store_scatter(cs_v, [pos], packed, mask=keep)
            plsc.store_scatter(ce0_v, [pos], ex0, mask=keep)
            plsc.store_scatter(ce1_v, [pos], ex1, mask=keep)
            return offset + jnp.sum(ki)

        count = lax.fori_loop(0, E_t // CHUNK_A, chunk_a, jnp.int32(0))

        pltpu.sync_copy(cs_v, src_sp.at[sid])
        pltpu.sync_copy(ce0_v, exh_sp.at[0, sid])
        pltpu.sync_copy(ce1_v, exh_sp.at[1, sid])
        cnt1_sm[...] = jnp.broadcast_to(count, (L,))
        pltpu.sync_copy(cnt1_sm, counts_sp.at[pl.ds(sid * L, L)])

        pltpu.sync_copy(den0_v, den_out.at[wid, 0])
        pltpu.sync_copy(den1_v, den_out.at[wid, 1])

        plsc.subcore_barrier()
        pltpu.sync_copy(counts_sp, cnts_sm)

        RPP = NRMAX // n_splits
        NTOT = NS * RPP

        def decode(k):
            p = jnp.minimum(k // RPP, NS - 1)
            r = (k % RPP) * n_splits + split
            valid = (k < NTOT) & (r * ROUND_E < cnts_sm[pl.ds(p * L, L)][0])
            return p, r, valid

        def fire(k, pkX, sidxX, exX, rowsX, semX):
            p, r, valid = decode(k)

            @pl.when(valid)
            def _():
                base = r * ROUND_E
                pltpu.sync_copy(src_sp.at[p, pl.ds(base, ROUND_E)], pkX)
                pltpu.sync_copy(exh_sp.at[hsel, p, pl.ds(base, ROUND_E)], exX)

                def unpack(u, _4):
                    usl = pl.ds(u * L, L)
                    sidxX[usl] = lax.shift_right_logical(pkX[usl], 10)
                    return 0
                lax.fori_loop(0, ROUND_E // L, unpack, 0)
                ht_src = ht_sp if stage_tab else ht_hbm
                for j in range(ROUND_E // 128):
                    pltpu.async_copy(
                        ht_src.at[group].at[sidxX.at[pl.ds(j * 128, 128)]],
                        rowsX.at[pl.ds(j * 128, 128)], semX)

        def drain(k, rowsX, semX):
            _, _, valid = decode(k)

            @pl.when(valid)
            def _():
                for j in range(ROUND_E // 128):
                    pltpu.make_async_copy(
                        ht_hbm.at[group, pl.ds(0, 128)],
                        rowsX.at[pl.ds(j * 128, 128)], semX).wait()

        def compute(k, pkX, exX, rowsX):
            _, _, valid = decode(k)

            @pl.when(valid)
            def _():
                def grp(g, _3):
                    exv = exX[pl.ds(g * L, L)]
                    dstv = pkX[pl.ds(g * L, L)] & 1023
                    ev0 = jnp.broadcast_to(g * L, (L,))
                    for l in range(L):
                        e = ev0 + l
                        s = jnp.broadcast_to(exv[l], (L,))
                        dl = jnp.broadcast_to(dstv[l], (L,))
                        for j in range(CPT):
                            cj = lane + j * L
                            v = plsc.load_gather(rowsX, [e, cj])
                            a = plsc.load_gather(accs_v[j], [dl, lane])
                            plsc.store_scatter(accs_v[j], [dl, lane],
                                               a + v * s)
                    return 0
                lax.fori_loop(0, ROUND_E // L, grp, 0)

        fire(jnp.int32(0), pk_r, sidx_r, ex_r, rows_r, sem)

        def pipe(m, _):
            ka = 2 * m
            kb = 2 * m + 1
            drain(ka, rows_r, sem)
            fire(kb, pk_b, sidx_b, ex_b, rows_b, sem_b)
            compute(ka, pk_r, ex_r, rows_r)
            drain(kb, rows_b, sem_b)
            fire(kb + 1, pk_r, sidx_r, ex_r, rows_r, sem)
            compute(kb, pk_b, ex_b, rows_b)
            return 0
        lax.fori_loop(0, (NTOT + 1) // 2, pipe, 0)

        for j in range(CPT):
            pltpu.sync_copy(
                accs_v[j],
                acc_out.at[cid * n_splits + split, :,
                           pl.ds(group * WT + j * L, L)])

    scratch = [
        pltpu.VMEM((n_src,), F32), pltpu.VMEM((n_src,), F32),
        pltpu.VMEM((keep_n,), F32), pltpu.VMEM((keep_n,), F32),
        pltpu.VMEM((keep_n,), F32), pltpu.VMEM((keep_n,), F32),
        pltpu.VMEM((DEN,), F32), pltpu.VMEM((DEN,), F32),
        pltpu.VMEM((CHUNK_A,), I32), pltpu.VMEM((CHUNK_A,), I32),
        pltpu.VMEM((CAP,), I32),
        pltpu.VMEM((CAP,), F32), pltpu.VMEM((CAP,), F32),
        pltpu.VMEM((ROUND_E,), I32), pltpu.VMEM((ROUND_E,), I32),
        pltpu.VMEM((ROUND_E,), F32),
        pltpu.VMEM((ROUND_E, WT), F32),
        pltpu.VMEM((ROUND_E,), I32), pltpu.VMEM((ROUND_E,), I32),
        pltpu.VMEM((ROUND_E,), F32),
        pltpu.VMEM((ROUND_E, WT), F32),
        pltpu.VMEM((ACC_R, L), F32), pltpu.VMEM((ACC_R, L), F32),
        pltpu.VMEM_SHARED((NS, CAP), I32),
        pltpu.VMEM_SHARED((2, NS, CAP), F32),
        pltpu.VMEM_SHARED((NS * L,), I32),
        pltpu.VMEM((L,), I32),
        pltpu.VMEM((NS * L,), I32),
        pltpu.SemaphoreType.DMA,
        pltpu.SemaphoreType.DMA,
    ]
    if stage_tab:
        scratch.append(pltpu.VMEM_SHARED((n_groups, n_src, WT), F32))
    out_type = [jax.ShapeDtypeStruct((G, ACC_R, width), F32),
                jax.ShapeDtypeStruct((NW, 2, DEN), F32)]
    return pl.kernel(
        body, out_type=out_type, mesh=mesh, scratch_types=scratch,
        compiler_params=pltpu.CompilerParams(needs_layout_passes=False,
                                             use_tc_tiling_on_sc=False),
    ), ACC_R, DEN


_edge0, ACC_R0, DEN0 = _make_edge_kernel(5000, 1000, 320000, HEADS * HIDDEN)
_edge1, ACC_R1, DEN1 = _make_edge_kernel(1000, 1000, 64000, HEADS * OUT_C)


def _tc_pre_body(x5, wcat, a0s, a0d, bsk, h0_ref, tsrc_ref, tdst_ref, skip_ref):
    H = jnp.dot(x5[...], wcat[...], preferred_element_type=F32)
    h0 = H[:, :HEADS * HIDDEN]
    hd = H[:, HEADS * HIDDEN:2 * HEADS * HIDDEN]
    for c in range((HEADS * HIDDEN) // (CPT * 16)):
        h0_ref[c] = h0[:, c * CPT * 16:(c + 1) * CPT * 16]
    skip_ref[...] = H[:1000, 2 * HEADS * HIDDEN:] + bsk[...]
    as0 = jnp.sum(h0[:, :HIDDEN] * a0s[0:1, :], axis=1)
    as1 = jnp.sum(h0[:, HIDDEN:] * a0s[1:2, :], axis=1)
    ad0 = jnp.sum(hd[:, :HIDDEN] * a0d[0:1, :], axis=1)
    ad1 = jnp.sum(hd[:, HIDDEN:] * a0d[1:2, :], axis=1)
    tsrc_ref[0:1, :] = as0[None, :]
    tsrc_ref[1:2, :] = as1[None, :]
    m0 = jnp.max(jnp.abs(as0))
    m1 = jnp.max(jnp.abs(as1))
    tdst_ref[0:1, :] = ad0[None, :1000]
    tdst_ref[1:2, :] = ad1[None, :1000]
    tdst_ref[2:3, :] = jnp.abs(ad0[None, :1000]) + m0
    tdst_ref[3:4, :] = jnp.abs(ad1[None, :1000]) + m1


def _tc_mid_body(acc, den, skip0, wcat1, a1s, a1d, bsk1,
                 h1_ref, tsrc_ref, tdst_ref, skip1_ref):
    d = jnp.sum(den[...], axis=0)
    inv0 = 1.0 / (d[0:1, :1000] + 1e-16)
    inv1 = 1.0 / (d[1:2, :1000] + 1e-16)
    accs = jnp.sum(acc[:, :1000, :], axis=0)
    out0 = jnp.concatenate(
        [accs[:, :HIDDEN] * inv0.T, accs[:, HIDDEN:] * inv1.T], axis=1)
    pre = out0 + skip0[...]
    h = jnp.where(pre > 0, pre, jnp.exp(pre) - 1.0)
    H1 = jnp.dot(h, wcat1[...], preferred_element_type=F32)
    h1 = H1[:, :HEADS * OUT_C]
    hd1 = H1[:, HEADS * OUT_C:2 * HEADS * OUT_C]
    for c in range((HEADS * OUT_C) // (CPT * 16)):
        h1_ref[c] = h1[:, c * CPT * 16:(c + 1) * CPT * 16]
    skip1_ref[...] = H1[:, 2 * HEADS * OUT_C:] + bsk1[...]
    as0 = jnp.sum(h1[:, :OUT_C] * a1s[0:1, :], axis=1)
    as1 = jnp.sum(h1[:, OUT_C:] * a1s[1:2, :], axis=1)
    ad0 = jnp.sum(hd1[:, :OUT_C] * a1d[0:1, :], axis=1)
    ad1 = jnp.sum(hd1[:, OUT_C:] * a1d[1:2, :], axis=1)
    tsrc_ref[0:1, :] = as0[None, :]
    tsrc_ref[1:2, :] = as1[None, :]
    m0 = jnp.max(jnp.abs(as0))
    m1 = jnp.max(jnp.abs(as1))
    tdst_ref[0:1, :] = ad0[None, :]
    tdst_ref[1:2, :] = ad1[None, :]
    tdst_ref[2:3, :] = jnp.abs(ad0[None, :]) + m0
    tdst_ref[3:4, :] = jnp.abs(ad1[None, :]) + m1


def _tc_post_body(acc, den, skip1, o_ref):
    d = jnp.sum(den[...], axis=0)
    inv0 = 1.0 / (d[0:1, :1000] + 1e-16)
    inv1 = 1.0 / (d[1:2, :1000] + 1e-16)
    accs = jnp.sum(acc[:, :1000, :], axis=0)
    m0 = accs[:, :OUT_C] * inv0.T
    m1 = accs[:, OUT_C:] * inv1.T
    o = 0.5 * (m0 + m1) + skip1[...]
    t = o - jnp.max(o, axis=1, keepdims=True)
    o_ref[...] = t - jnp.log(jnp.sum(jnp.exp(t), axis=1, keepdims=True))


def kernel(x, edge_index0, edge_index1, n_target0, n_target1,
           W0_src, W0_dst, a0_src, a0_dst, Wskip0, bskip0,
           W1_src, W1_dst, a1_src, a1_dst, Wskip1, bskip1):
    x5 = x[:5000]
    wcat0 = jnp.concatenate([W0_src, W0_dst, Wskip0], axis=1)
    wcat1 = jnp.concatenate([W1_src, W1_dst, Wskip1], axis=1)

    ht0, tsrc0, tdst0, skip0 = pl.pallas_call(
        _tc_pre_body,
        out_shape=[jax.ShapeDtypeStruct(
                       ((HEADS * HIDDEN) // (CPT * 16), 5000, CPT * 16), F32),
                   jax.ShapeDtypeStruct((2, 5000), F32),
                   jax.ShapeDtypeStruct((4, 1000), F32),
                   jax.ShapeDtypeStruct((1000, HEADS * HIDDEN), F32)],
    )(x5, wcat0, a0_src, a0_dst, bskip0.reshape(1, -1))

    acc0, den0 = _edge0(edge_index0[0], edge_index0[1], ht0, tsrc0, tdst0)

    ht1, tsrc1, tdst1, skip1 = pl.pallas_call(
        _tc_mid_body,
        out_shape=[jax.ShapeDtypeStruct(
                       ((HEADS * OUT_C) // (CPT * 16), 1000, CPT * 16), F32),
                   jax.ShapeDtypeStruct((2, 1000), F32),
                   jax.ShapeDtypeStruct((4, 1000), F32),
                   jax.ShapeDtypeStruct((1000, OUT_C), F32)],
    )(acc0, den0, skip0, wcat1, a1_src, a1_dst, bskip1.reshape(1, -1))

    acc1, den1 = _edge1(edge_index1[0], edge_index1[1], ht1, tsrc1, tdst1)

    return pl.pallas_call(
        _tc_post_body,
        out_shape=jax.ShapeDtypeStruct((1000, OUT_C), F32),
    )(acc1, den1, skip1)


# speedup vs baseline: 1.0546x; 1.0546x over previous
import functools
import jax
import jax.numpy as jnp
from jax import lax
from jax.experimental import pallas as pl
from jax.experimental.pallas import tpu as pltpu
from jax.experimental.pallas import tpu_sc as plsc

HEADS = 2
HIDDEN = 128
OUT_C = 64
D_IN = 128

NC, NS, L = 2, 16, 16
NW = NC * NS

F32 = jnp.float32
I32 = jnp.int32


ROUND_E = 256
CHUNK_A = 400
CPT = 2


def _make_edge_kernel(n_src, keep_n, E, width):
    E_t = E // NW
    CAP = ((E_t + ROUND_E - 1) // ROUND_E) * ROUND_E
    NRMAX = CAP // ROUND_E
    DEN = ((keep_n + 15) // 16) * 16
    ACC_R = ((keep_n + 15) // 16) * 16
    n_chunks = width // L
    n_groups = n_chunks // CPT
    n_splits = NS // n_groups
    G = NC * n_splits
    half_chunks = n_chunks // 2
    WT = CPT * L

    stage_tab = n_src * width * 4 <= 1_000_000
    rows_per = n_src // n_splits
    CHA = E_t if E_t <= 2000 else CHUNK_A
    NCH = E_t // CHA

    mesh = plsc.VectorSubcoreMesh(core_axis_name="c", subcore_axis_name="s",
                                  num_cores=NC, num_subcores=NS)

    def body(esrc, edst, ht_hbm, tabs_src, tabs_dst,
             acc_out, den_out,
             as0_v, as1_v, ad0_v, ad1_v, c0_v, c1_v,
             den0_v, den1_v, es_v, ed_v, es_b, ed_b,
             cs_v, ce0_v, ce1_v,
             pk_r, sidx_r, ex_r, rows_r,
             pk_b, sidx_b, ex_b, rows_b, acc_v0, acc_v1,
             src_sp, exh_sp, counts_sp,
             cnt1_sm, cnts_sm, sem, sem_b, *maybe_ht_sp):
        accs_v = (acc_v0, acc_v1)
        ht_sp = maybe_ht_sp[0] if stage_tab else None
        cid = lax.axis_index("c")
        sid = lax.axis_index("s")
        wid = sid * NC + cid
        group = sid % n_groups
        split = sid // n_groups
        hsel = (group * CPT) // half_chunks

        pltpu.sync_copy(tabs_src.at[0], as0_v)
        pltpu.sync_copy(tabs_src.at[1], as1_v)
        pltpu.sync_copy(tabs_dst.at[0], ad0_v)
        pltpu.sync_copy(tabs_dst.at[1], ad1_v)
        pltpu.sync_copy(tabs_dst.at[2], c0_v)
        pltpu.sync_copy(tabs_dst.at[3], c1_v)
        if stage_tab:
            gi = sid % n_groups
            ro = (sid // n_groups) * rows_per
            pltpu.sync_copy(ht_hbm.at[gi, pl.ds(ro, rows_per)],
                            ht_sp.at[gi, pl.ds(ro, rows_per)])

        zf = jnp.zeros((L,), F32)
        zi = jnp.zeros((L,), I32)
        lane = lax.iota(I32, L)

        def zero_body(i, _):
            sl = pl.ds(i * L, L)
            cs_v[sl] = zi
            ce0_v[sl] = zf
            ce1_v[sl] = zf
            return 0
        lax.fori_loop(0, CAP // L, zero_body, 0)

        def zero_den(i, _):
            sl = pl.ds(i * L, L)
            den0_v[sl] = zf
            den1_v[sl] = zf
            return 0
        lax.fori_loop(0, DEN // L, zero_den, 0)

        def zero_acc(i, _):
            iv = jnp.broadcast_to(i, (L,))
            for j in range(CPT):
                plsc.store_scatter(accs_v[j], [iv, lane], zf)
            return 0
        lax.fori_loop(0, ACC_R, zero_acc, 0)

        def fire_chunk(c, esX, edX, semX):
            @pl.when(c < NCH)
            def _():
                off = wid * E_t + c * CHA
                pltpu.async_copy(esrc.at[pl.ds(off, CHA)], esX, semX)
                pltpu.async_copy(edst.at[pl.ds(off, CHA)], edX, semX)

        def drain_chunk(esX, edX, semX):
            pltpu.make_async_copy(esrc.at[pl.ds(0, CHA)], esX, semX).wait()
            pltpu.make_async_copy(esrc.at[pl.ds(0, CHA)], edX, semX).wait()

        def proc_chunk(esX, edX, offset0):
            def pa(i, offset):
                return phase_a(i, offset, esX, edX)
            return lax.fori_loop(0, CHA // L, pa, offset0)

        def phase_a(i, offset, es_v, ed_v):
            sl = pl.ds(i * L, L)
            src16 = es_v[sl]
            dst16 = ed_v[sl]
            keep = dst16 < keep_n
            dstc = jnp.where(keep, dst16, 0)
            a_s0 = plsc.load_gather(as0_v, [src16])
            a_s1 = plsc.load_gather(as1_v, [src16])
            a_d0 = plsc.load_gather(ad0_v, [dstc])
            a_d1 = plsc.load_gather(ad1_v, [dstc])
            cc0 = plsc.load_gather(c0_v, [dstc])
            cc1 = plsc.load_gather(c1_v, [dstc])
            z0 = a_s0 + a_d0
            z1 = a_s1 + a_d1
            al0 = jnp.where(z0 >= 0, z0, 0.2 * z0)
            al1 = jnp.where(z1 >= 0, z1, 0.2 * z1)
            ex0 = jnp.exp(al0 - cc0)
            ex1 = jnp.exp(al1 - cc1)
            plsc.addupdate_scatter(den0_v, [dstc], ex0, mask=keep)
            plsc.addupdate_scatter(den1_v, [dstc], ex1, mask=keep)
            ki = keep.astype(I32)
            pos = offset + plsc.cumsum(ki) - ki
            packed = lax.shift_left(src16, 10) | dstc
            plsc.store_scatter(cs_v, [pos], packed, mask=keep)
            plsc.store_scatter(ce0_v, [pos], ex0, mask=keep)
            plsc.store_scatter(ce1_v, [pos], ex1, mask=keep)
            return offset + jnp.sum(ki)

        fire_chunk(jnp.int32(0), es_v, ed_v, sem)

        def chunk_pair(m, offset):
            ca = 2 * m
            cb = 2 * m + 1
            drain_chunk(es_v, ed_v, sem)
            fire_chunk(cb, es_b, ed_b, sem_b)
            offset = proc_chunk(es_v, ed_v, offset)

            @pl.when(cb < NCH)
            def _():
                drain_chunk(es_b, ed_b, sem_b)
                fire_chunk(cb + 1, es_v, ed_v, sem)
            return lax.cond(cb < NCH,
                            lambda o: proc_chunk(es_b, ed_b, o),
                            lambda o: o, offset)
        count = lax.fori_loop(0, (NCH + 1) // 2, chunk_pair, jnp.int32(0))

        pltpu.sync_copy(cs_v, src_sp.at[sid])
        pltpu.sync_copy(ce0_v, exh_sp.at[0, sid])
        pltpu.sync_copy(ce1_v, exh_sp.at[1, sid])
        cnt1_sm[...] = jnp.broadcast_to(count, (L,))
        pltpu.sync_copy(cnt1_sm, counts_sp.at[pl.ds(sid * L, L)])

        pltpu.sync_copy(den0_v, den_out.at[wid, 0])
        pltpu.sync_copy(den1_v, den_out.at[wid, 1])

        plsc.subcore_barrier()
        pltpu.sync_copy(counts_sp, cnts_sm)

        RPP = NRMAX // n_splits
        NTOT = NS * RPP

        def decode(k):
            p = jnp.minimum(k // RPP, NS - 1)
            r = (k % RPP) * n_splits + split
            valid = (k < NTOT) & (r * ROUND_E < cnts_sm[pl.ds(p * L, L)][0])
            return p, r, valid

        def fire(k, pkX, sidxX, exX, rowsX, semX):
            p, r, valid = decode(k)

            @pl.when(valid)
            def _():
                base = r * ROUND_E
                pltpu.sync_copy(src_sp.at[p, pl.ds(base, ROUND_E)], pkX)
                pltpu.sync_copy(exh_sp.at[hsel, p, pl.ds(base, ROUND_E)], exX)

                def unpack(u, _4):
                    usl = pl.ds(u * L, L)
                    sidxX[usl] = lax.shift_right_logical(pkX[usl], 10)
                    return 0
                lax.fori_loop(0, ROUND_E // L, unpack, 0)
                ht_src = ht_sp if stage_tab else ht_hbm
                for j in range(ROUND_E // 128):
                    pltpu.async_copy(
                        ht_src.at[group].at[sidxX.at[pl.ds(j * 128, 128)]],
                        rowsX.at[pl.ds(j * 128, 128)], semX)

        def drain(k, rowsX, semX):
            _, _, valid = decode(k)

            @pl.when(valid)
            def _():
                for j in range(ROUND_E // 128):
                    pltpu.make_async_copy(
                        ht_hbm.at[group, pl.ds(0, 128)],
                        rowsX.at[pl.ds(j * 128, 128)], semX).wait()

        def compute(k, pkX, exX, rowsX):
            _, _, valid = decode(k)

            @pl.when(valid)
            def _():
                def grp(g, _3):
                    exv = exX[pl.ds(g * L, L)]
                    dstv = pkX[pl.ds(g * L, L)] & 1023
                    ev0 = jnp.broadcast_to(g * L, (L,))
                    for l in range(L):
                        e = ev0 + l
                        s = jnp.broadcast_to(exv[l], (L,))
                        dl = jnp.broadcast_to(dstv[l], (L,))
                        for j in range(CPT):
                            cj = lane + j * L
                            v = plsc.load_gather(rowsX, [e, cj])
                            a = plsc.load_gather(accs_v[j], [dl, lane])
                            plsc.store_scatter(accs_v[j], [dl, lane],
                                               a + v * s)
                    return 0
                lax.fori_loop(0, ROUND_E // L, grp, 0)

        fire(jnp.int32(0), pk_r, sidx_r, ex_r, rows_r, sem)

        def pipe(m, _):
            ka = 2 * m
            kb = 2 * m + 1
            drain(ka, rows_r, sem)
            fire(kb, pk_b, sidx_b, ex_b, rows_b, sem_b)
            compute(ka, pk_r, ex_r, rows_r)
            drain(kb, rows_b, sem_b)
            fire(kb + 1, pk_r, sidx_r, ex_r, rows_r, sem)
            compute(kb, pk_b, ex_b, rows_b)
            return 0
        lax.fori_loop(0, (NTOT + 1) // 2, pipe, 0)

        for j in range(CPT):
            pltpu.sync_copy(
                accs_v[j],
                acc_out.at[cid * n_splits + split, :,
                           pl.ds(group * WT + j * L, L)])

    scratch = [
        pltpu.VMEM((n_src,), F32), pltpu.VMEM((n_src,), F32),
        pltpu.VMEM((keep_n,), F32), pltpu.VMEM((keep_n,), F32),
        pltpu.VMEM((keep_n,), F32), pltpu.VMEM((keep_n,), F32),
        pltpu.VMEM((DEN,), F32), pltpu.VMEM((DEN,), F32),
        pltpu.VMEM((CHA,), I32), pltpu.VMEM((CHA,), I32),
        pltpu.VMEM((CHA,), I32), pltpu.VMEM((CHA,), I32),
        pltpu.VMEM((CAP,), I32),
        pltpu.VMEM((CAP,), F32), pltpu.VMEM((CAP,), F32),
        pltpu.VMEM((ROUND_E,), I32), pltpu.VMEM((ROUND_E,), I32),
        pltpu.VMEM((ROUND_E,), F32),
        pltpu.VMEM((ROUND_E, WT), F32),
        pltpu.VMEM((ROUND_E,), I32), pltpu.VMEM((ROUND_E,), I32),
        pltpu.VMEM((ROUND_E,), F32),
        pltpu.VMEM((ROUND_E, WT), F32),
        pltpu.VMEM((ACC_R, L), F32), pltpu.VMEM((ACC_R, L), F32),
        pltpu.VMEM_SHARED((NS, CAP), I32),
        pltpu.VMEM_SHARED((2, NS, CAP), F32),
        pltpu.VMEM_SHARED((NS * L,), I32),
        pltpu.VMEM((L,), I32),
        pltpu.VMEM((NS * L,), I32),
        pltpu.SemaphoreType.DMA,
        pltpu.SemaphoreType.DMA,
    ]
    if stage_tab:
        scratch.append(pltpu.VMEM_SHARED((n_groups, n_src, WT), F32))
    out_type = [jax.ShapeDtypeStruct((G, ACC_R, width), F32),
                jax.ShapeDtypeStruct((NW, 2, DEN), F32)]
    return pl.kernel(
        body, out_type=out_type, mesh=mesh, scratch_types=scratch,
        compiler_params=pltpu.CompilerParams(needs_layout_passes=False,
                                             use_tc_tiling_on_sc=False),
    ), ACC_R, DEN


_edge0, ACC_R0, DEN0 = _make_edge_kernel(5000, 1000, 320000, HEADS * HIDDEN)
_edge1, ACC_R1, DEN1 = _make_edge_kernel(1000, 1000, 64000, HEADS * OUT_C)


def _tc_pre_body(x5, wcat, a0s, a0d, bsk, h0_ref, tsrc_ref, tdst_ref, skip_ref):
    H = jnp.dot(x5[...], wcat[...], preferred_element_type=F32)
    h0 = H[:, :HEADS * HIDDEN]
    hd = H[:, HEADS * HIDDEN:2 * HEADS * HIDDEN]
    for c in range((HEADS * HIDDEN) // (CPT * 16)):
        h0_ref[c] = h0[:, c * CPT * 16:(c + 1) * CPT * 16]
    skip_ref[...] = H[:1000, 2 * HEADS * HIDDEN:] + bsk[...]
    as0 = jnp.sum(h0[:, :HIDDEN] * a0s[0:1, :], axis=1)
    as1 = jnp.sum(h0[:, HIDDEN:] * a0s[1:2, :], axis=1)
    ad0 = jnp.sum(hd[:, :HIDDEN] * a0d[0:1, :], axis=1)
    ad1 = jnp.sum(hd[:, HIDDEN:] * a0d[1:2, :], axis=1)
    tsrc_ref[0:1, :] = as0[None, :]
    tsrc_ref[1:2, :] = as1[None, :]
    m0 = jnp.max(jnp.abs(as0))
    m1 = jnp.max(jnp.abs(as1))
    tdst_ref[0:1, :] = ad0[None, :1000]
    tdst_ref[1:2, :] = ad1[None, :1000]
    tdst_ref[2:3, :] = jnp.abs(ad0[None, :1000]) + m0
    tdst_ref[3:4, :] = jnp.abs(ad1[None, :1000]) + m1


def _tc_mid_body(acc, den, skip0, wcat1, a1s, a1d, bsk1,
                 h1_ref, tsrc_ref, tdst_ref, skip1_ref):
    d = jnp.sum(den[...], axis=0)
    inv0 = 1.0 / (d[0:1, :1000] + 1e-16)
    inv1 = 1.0 / (d[1:2, :1000] + 1e-16)
    accs = jnp.sum(acc[:, :1000, :], axis=0)
    out0 = jnp.concatenate(
        [accs[:, :HIDDEN] * inv0.T, accs[:, HIDDEN:] * inv1.T], axis=1)
    pre = out0 + skip0[...]
    h = jnp.where(pre > 0, pre, jnp.exp(pre) - 1.0)
    H1 = jnp.dot(h, wcat1[...], preferred_element_type=F32)
    h1 = H1[:, :HEADS * OUT_C]
    hd1 = H1[:, HEADS * OUT_C:2 * HEADS * OUT_C]
    for c in range((HEADS * OUT_C) // (CPT * 16)):
        h1_ref[c] = h1[:, c * CPT * 16:(c + 1) * CPT * 16]
    skip1_ref[...] = H1[:, 2 * HEADS * OUT_C:] + bsk1[...]
    as0 = jnp.sum(h1[:, :OUT_C] * a1s[0:1, :], axis=1)
    as1 = jnp.sum(h1[:, OUT_C:] * a1s[1:2, :], axis=1)
    ad0 = jnp.sum(hd1[:, :OUT_C] * a1d[0:1, :], axis=1)
    ad1 = jnp.sum(hd1[:, OUT_C:] * a1d[1:2, :], axis=1)
    tsrc_ref[0:1, :] = as0[None, :]
    tsrc_ref[1:2, :] = as1[None, :]
    m0 = jnp.max(jnp.abs(as0))
    m1 = jnp.max(jnp.abs(as1))
    tdst_ref[0:1, :] = ad0[None, :]
    tdst_ref[1:2, :] = ad1[None, :]
    tdst_ref[2:3, :] = jnp.abs(ad0[None, :]) + m0
    tdst_ref[3:4, :] = jnp.abs(ad1[None, :]) + m1


def _tc_post_body(acc, den, skip1, o_ref):
    d = jnp.sum(den[...], axis=0)
    inv0 = 1.0 / (d[0:1, :1000] + 1e-16)
    inv1 = 1.0 / (d[1:2, :1000] + 1e-16)
    accs = jnp.sum(acc[:, :1000, :], axis=0)
    m0 = accs[:, :OUT_C] * inv0.T
    m1 = accs[:, OUT_C:] * inv1.T
    o = 0.5 * (m0 + m1) + skip1[...]
    t = o - jnp.max(o, axis=1, keepdims=True)
    o_ref[...] = t - jnp.log(jnp.sum(jnp.exp(t), axis=1, keepdims=True))


def kernel(x, edge_index0, edge_index1, n_target0, n_target1,
           W0_src, W0_dst, a0_src, a0_dst, Wskip0, bskip0,
           W1_src, W1_dst, a1_src, a1_dst, Wskip1, bskip1):
    x5 = x[:5000]
    wcat0 = jnp.concatenate([W0_src, W0_dst, Wskip0], axis=1)
    wcat1 = jnp.concatenate([W1_src, W1_dst, Wskip1], axis=1)

    ht0, tsrc0, tdst0, skip0 = pl.pallas_call(
        _tc_pre_body,
        out_shape=[jax.ShapeDtypeStruct(
                       ((HEADS * HIDDEN) // (CPT * 16), 5000, CPT * 16), F32),
                   jax.ShapeDtypeStruct((2, 5000), F32),
                   jax.ShapeDtypeStruct((4, 1000), F32),
                   jax.ShapeDtypeStruct((1000, HEADS * HIDDEN), F32)],
    )(x5, wcat0, a0_src, a0_dst, bskip0.reshape(1, -1))

    acc0, den0 = _edge0(edge_index0[0], edge_index0[1], ht0, tsrc0, tdst0)

    ht1, tsrc1, tdst1, skip1 = pl.pallas_call(
        _tc_mid_body,
        out_shape=[jax.ShapeDtypeStruct(
                       ((HEADS * OUT_C) // (CPT * 16), 1000, CPT * 16), F32),
                   jax.ShapeDtypeStruct((2, 1000), F32),
                   jax.ShapeDtypeStruct((4, 1000), F32),
                   jax.ShapeDtypeStruct((1000, OUT_C), F32)],
    )(acc0, den0, skip0, wcat1, a1_src, a1_dst, bskip1.reshape(1, -1))

    acc1, den1 = _edge1(edge_index1[0], edge_index1[1], ht1, tsrc1, tdst1)

    return pl.pallas_call(
        _tc_post_body,
        out_shape=jax.ShapeDtypeStruct((1000, OUT_C), F32),
    )(acc1, den1, skip1)


# speedup vs baseline: 1.0580x; 1.0032x over previous
import jax
import jax.numpy as jnp
from jax import lax
from jax.experimental import pallas as pl
from jax.experimental.pallas import tpu as pltpu
from jax.experimental.pallas import tpu_sc as plsc

HEADS = 2
HIDDEN = 128
OUT_C = 64
D_IN = 128

NC, NS, L = 2, 16, 16
NW = NC * NS

F32 = jnp.float32
I32 = jnp.int32


ROUND_E = 256
CHUNK_A = 400
CPT = 2


def _make_edge_kernel(n_src, keep_n, E, width):
    E_t = E // NW
    CAP = ((E_t + ROUND_E - 1) // ROUND_E) * ROUND_E
    NRMAX = CAP // ROUND_E
    DEN = ((keep_n + 15) // 16) * 16
    ACC_R = ((keep_n + 15) // 16) * 16
    n_chunks = width // L
    n_groups = n_chunks // CPT
    n_splits = NS // n_groups
    G = NC * n_splits
    half_chunks = n_chunks // 2
    WT = CPT * L

    stage_tab = n_src * width * 4 <= 1_000_000
    rows_per = n_src // n_splits
    CHA = E_t if E_t <= 2000 else CHUNK_A
    NCH = E_t // CHA

    mesh = plsc.VectorSubcoreMesh(core_axis_name="c", subcore_axis_name="s",
                                  num_cores=NC, num_subcores=NS)

    def body(esrc, edst, ht_hbm, tabs_src, tabs_dst,
             acc_out, den_out,
             as0_v, as1_v, ad0_v, ad1_v, c0_v, c1_v,
             den0_v, den1_v, es_v, ed_v, es_b, ed_b,
             cs_v, ce0_v, ce1_v,
             pk_r, sidx_r, ex_r, rows_r,
             pk_b, sidx_b, ex_b, rows_b, acc_v0, acc_v1,
             src_sp, exh_sp, counts_sp,
             cnt1_sm, cnts_sm, sem, sem_b, *maybe_ht_sp):
        accs_v = (acc_v0, acc_v1)
        ht_sp = maybe_ht_sp[0] if stage_tab else None
        cid = lax.axis_index("c")
        sid = lax.axis_index("s")
        wid = sid * NC + cid
        group = sid % n_groups
        split = sid // n_groups
        hsel = (group * CPT) // half_chunks

        pltpu.sync_copy(tabs_src.at[0], as0_v)
        pltpu.sync_copy(tabs_src.at[1], as1_v)
        pltpu.sync_copy(tabs_dst.at[0], ad0_v)
        pltpu.sync_copy(tabs_dst.at[1], ad1_v)
        pltpu.sync_copy(tabs_dst.at[2], c0_v)
        pltpu.sync_copy(tabs_dst.at[3], c1_v)
        if stage_tab:
            gi = sid % n_groups
            ro = (sid // n_groups) * rows_per
            pltpu.sync_copy(ht_hbm.at[gi, pl.ds(ro, rows_per)],
                            ht_sp.at[gi, pl.ds(ro, rows_per)])

        zf = jnp.zeros((L,), F32)
        zi = jnp.zeros((L,), I32)
        lane = lax.iota(I32, L)

        def zero_body(i, _):
            sl = pl.ds(i * L, L)
            cs_v[sl] = zi
            ce0_v[sl] = zf
            ce1_v[sl] = zf
            return 0
        lax.fori_loop(0, CAP // L, zero_body, 0)

        def zero_den(i, _):
            sl = pl.ds(i * L, L)
            den0_v[sl] = zf
            den1_v[sl] = zf
            return 0
        lax.fori_loop(0, DEN // L, zero_den, 0)

        def zero_acc(i, _):
            iv = jnp.broadcast_to(i, (L,))
            for j in range(CPT):
                plsc.store_scatter(accs_v[j], [iv, lane], zf)
            return 0
        lax.fori_loop(0, ACC_R, zero_acc, 0)

        def fire_chunk(c, esX, edX, semX):
            @pl.when(c < NCH)
            def _():
                off = wid * E_t + c * CHA
                pltpu.async_copy(esrc.at[pl.ds(off, CHA)], esX, semX)
                pltpu.async_copy(edst.at[pl.ds(off, CHA)], edX, semX)

        def drain_chunk(esX, edX, semX):
            pltpu.make_async_copy(esrc.at[pl.ds(0, CHA)], esX, semX).wait()
            pltpu.make_async_copy(esrc.at[pl.ds(0, CHA)], edX, semX).wait()

        def proc_chunk(esX, edX, offset0):
            def pa(i, offset):
                return phase_a(i, offset, esX, edX)
            return lax.fori_loop(0, CHA // L, pa, offset0)

        def phase_a(i, offset, es_v, ed_v):
            sl = pl.ds(i * L, L)
            src16 = es_v[sl]
            dst16 = ed_v[sl]
            keep = dst16 < keep_n
            dstc = jnp.where(keep, dst16, 0)
            a_s0 = plsc.load_gather(as0_v, [src16])
            a_s1 = plsc.load_gather(as1_v, [src16])
            a_d0 = plsc.load_gather(ad0_v, [dstc])
            a_d1 = plsc.load_gather(ad1_v, [dstc])
            cc0 = plsc.load_gather(c0_v, [dstc])
            cc1 = plsc.load_gather(c1_v, [dstc])
            z0 = a_s0 + a_d0
            z1 = a_s1 + a_d1
            al0 = jnp.where(z0 >= 0, z0, 0.2 * z0)
            al1 = jnp.where(z1 >= 0, z1, 0.2 * z1)
            ex0 = jnp.exp(al0 - cc0)
            ex1 = jnp.exp(al1 - cc1)
            plsc.addupdate_scatter(den0_v, [dstc], ex0, mask=keep)
            plsc.addupdate_scatter(den1_v, [dstc], ex1, mask=keep)
            ki = keep.astype(I32)
            pos = offset + plsc.cumsum(ki) - ki
            packed = lax.shift_left(src16, 10) | dstc
            plsc.store_scatter(cs_v, [pos], packed, mask=keep)
            plsc.store_scatter(ce0_v, [pos], ex0, mask=keep)
            plsc.store_scatter(ce1_v, [pos], ex1, mask=keep)
            return offset + jnp.sum(ki)

        fire_chunk(jnp.int32(0), es_v, ed_v, sem)

        def chunk_pair(m, offset):
            ca = 2 * m
            cb = 2 * m + 1
            drain_chunk(es_v, ed_v, sem)
            fire_chunk(cb, es_b, ed_b, sem_b)
            offset = proc_chunk(es_v, ed_v, offset)

            @pl.when(cb < NCH)
            def _():
                drain_chunk(es_b, ed_b, sem_b)
                fire_chunk(cb + 1, es_v, ed_v, sem)
            return lax.cond(cb < NCH,
                            lambda o: proc_chunk(es_b, ed_b, o),
                            lambda o: o, offset)
        count = lax.fori_loop(0, (NCH + 1) // 2, chunk_pair, jnp.int32(0))

        pltpu.sync_copy(cs_v, src_sp.at[sid])
        pltpu.sync_copy(ce0_v, exh_sp.at[0, sid])
        pltpu.sync_copy(ce1_v, exh_sp.at[1, sid])
        cnt1_sm[...] = jnp.broadcast_to(count, (L,))
        pltpu.sync_copy(cnt1_sm, counts_sp.at[pl.ds(sid * L, L)])

        pltpu.sync_copy(den0_v, den_out.at[wid, 0])
        pltpu.sync_copy(den1_v, den_out.at[wid, 1])

        plsc.subcore_barrier()
        pltpu.sync_copy(counts_sp, cnts_sm)

        RPP = NRMAX // n_splits
        NTOT = NS * RPP

        def decode(k):
            p = jnp.minimum(k // RPP, NS - 1)
            r = (k % RPP) * n_splits + split
            valid = (k < NTOT) & (r * ROUND_E < cnts_sm[pl.ds(p * L, L)][0])
            return p, r, valid

        def fire(k, pkX, sidxX, exX, rowsX, semX):
            p, r, valid = decode(k)

            @pl.when(valid)
            def _():
                base = r * ROUND_E
                pltpu.sync_copy(src_sp.at[p, pl.ds(base, ROUND_E)], pkX)
                pltpu.sync_copy(exh_sp.at[hsel, p, pl.ds(base, ROUND_E)], exX)

                def unpack(u, _4):
                    usl = pl.ds(u * L, L)
                    sidxX[usl] = lax.shift_right_logical(pkX[usl], 10)
                    return 0
                lax.fori_loop(0, ROUND_E // L, unpack, 0)
                ht_src = ht_sp if stage_tab else ht_hbm
                for j in range(ROUND_E // 128):
                    pltpu.async_copy(
                        ht_src.at[group].at[sidxX.at[pl.ds(j * 128, 128)]],
                        rowsX.at[pl.ds(j * 128, 128)], semX)

        def drain(k, rowsX, semX):
            _, _, valid = decode(k)

            @pl.when(valid)
            def _():
                for j in range(ROUND_E // 128):
                    pltpu.make_async_copy(
                        ht_hbm.at[group, pl.ds(0, 128)],
                        rowsX.at[pl.ds(j * 128, 128)], semX).wait()

        def compute(k, pkX, exX, rowsX):
            _, _, valid = decode(k)

            @pl.when(valid)
            def _():
                def grp(g, _3):
                    exv = exX[pl.ds(g * L, L)]
                    dstv = pkX[pl.ds(g * L, L)] & 1023
                    ev0 = jnp.broadcast_to(g * L, (L,))
                    for l in range(L):
                        e = ev0 + l
                        s = jnp.broadcast_to(exv[l], (L,))
                        dl = jnp.broadcast_to(dstv[l], (L,))
                        for j in range(CPT):
                            cj = lane + j * L
                            v = plsc.load_gather(rowsX, [e, cj])
                            a = plsc.load_gather(accs_v[j], [dl, lane])
                            plsc.store_scatter(accs_v[j], [dl, lane],
                                               a + v * s)
                    return 0
                lax.fori_loop(0, ROUND_E // L, grp, 0)

        fire(jnp.int32(0), pk_r, sidx_r, ex_r, rows_r, sem)

        def pipe(m, _):
            ka = 2 * m
            kb = 2 * m + 1
            drain(ka, rows_r, sem)
            fire(kb, pk_b, sidx_b, ex_b, rows_b, sem_b)
            compute(ka, pk_r, ex_r, rows_r)
            drain(kb, rows_b, sem_b)
            fire(kb + 1, pk_r, sidx_r, ex_r, rows_r, sem)
            compute(kb, pk_b, ex_b, rows_b)
            return 0
        lax.fori_loop(0, (NTOT + 1) // 2, pipe, 0)

        for j in range(CPT):
            pltpu.sync_copy(
                accs_v[j],
                acc_out.at[cid * n_splits + split, :,
                           pl.ds(group * WT + j * L, L)])

    scratch = [
        pltpu.VMEM((n_src,), F32), pltpu.VMEM((n_src,), F32),
        pltpu.VMEM((keep_n,), F32), pltpu.VMEM((keep_n,), F32),
        pltpu.VMEM((keep_n,), F32), pltpu.VMEM((keep_n,), F32),
        pltpu.VMEM((DEN,), F32), pltpu.VMEM((DEN,), F32),
        pltpu.VMEM((CHA,), I32), pltpu.VMEM((CHA,), I32),
        pltpu.VMEM((CHA,), I32), pltpu.VMEM((CHA,), I32),
        pltpu.VMEM((CAP,), I32),
        pltpu.VMEM((CAP,), F32), pltpu.VMEM((CAP,), F32),
        pltpu.VMEM((ROUND_E,), I32), pltpu.VMEM((ROUND_E,), I32),
        pltpu.VMEM((ROUND_E,), F32),
        pltpu.VMEM((ROUND_E, WT), F32),
        pltpu.VMEM((ROUND_E,), I32), pltpu.VMEM((ROUND_E,), I32),
        pltpu.VMEM((ROUND_E,), F32),
        pltpu.VMEM((ROUND_E, WT), F32),
        pltpu.VMEM((ACC_R, L), F32), pltpu.VMEM((ACC_R, L), F32),
        pltpu.VMEM_SHARED((NS, CAP), I32),
        pltpu.VMEM_SHARED((2, NS, CAP), F32),
        pltpu.VMEM_SHARED((NS * L,), I32),
        pltpu.VMEM((L,), I32),
        pltpu.VMEM((NS * L,), I32),
        pltpu.SemaphoreType.DMA,
        pltpu.SemaphoreType.DMA,
    ]
    if stage_tab:
        scratch.append(pltpu.VMEM_SHARED((n_groups, n_src, WT), F32))
    out_type = [jax.ShapeDtypeStruct((G, ACC_R, width), F32),
                jax.ShapeDtypeStruct((NW, 2, DEN), F32)]
    return pl.kernel(
        body, out_type=out_type, mesh=mesh, scratch_types=scratch,
        compiler_params=pltpu.CompilerParams(needs_layout_passes=False,
                                             use_tc_tiling_on_sc=False),
    ), ACC_R, DEN


_edge0, ACC_R0, DEN0 = _make_edge_kernel(5000, 1000, 320000, HEADS * HIDDEN)
_edge1, ACC_R1, DEN1 = _make_edge_kernel(1000, 1000, 64000, HEADS * OUT_C)


def _tc_pre_body(x5, wcat, a0s, a0d, bsk, h0_ref, tsrc_ref, tdst_ref, skip_ref):
    H = jnp.dot(x5[...], wcat[...], preferred_element_type=F32)
    h0 = H[:, :HEADS * HIDDEN]
    hd = H[:, HEADS * HIDDEN:2 * HEADS * HIDDEN]
    for c in range((HEADS * HIDDEN) // (CPT * 16)):
        h0_ref[c] = h0[:, c * CPT * 16:(c + 1) * CPT * 16]
    skip_ref[...] = H[:1000, 2 * HEADS * HIDDEN:] + bsk[...]
    as0 = jnp.sum(h0[:, :HIDDEN] * a0s[0:1, :], axis=1)
    as1 = jnp.sum(h0[:, HIDDEN:] * a0s[1:2, :], axis=1)
    ad0 = jnp.sum(hd[:, :HIDDEN] * a0d[0:1, :], axis=1)
    ad1 = jnp.sum(hd[:, HIDDEN:] * a0d[1:2, :], axis=1)
    tsrc_ref[0:1, :] = as0[None, :]
    tsrc_ref[1:2, :] = as1[None, :]
    m0 = jnp.max(jnp.abs(as0))
    m1 = jnp.max(jnp.abs(as1))
    tdst_ref[0:1, :] = ad0[None, :1000]
    tdst_ref[1:2, :] = ad1[None, :1000]
    tdst_ref[2:3, :] = jnp.abs(ad0[None, :1000]) + m0
    tdst_ref[3:4, :] = jnp.abs(ad1[None, :1000]) + m1


def _tc_mid_body(acc, den, skip0, wcat1, a1s, a1d, bsk1,
                 h1_ref, tsrc_ref, tdst_ref, skip1_ref):
    d = jnp.sum(den[...], axis=0)
    inv0 = 1.0 / (d[0:1, :1000] + 1e-16)
    inv1 = 1.0 / (d[1:2, :1000] + 1e-16)
    accs = jnp.sum(acc[:, :1000, :], axis=0)
    out0 = jnp.concatenate(
        [accs[:, :HIDDEN] * inv0.T, accs[:, HIDDEN:] * inv1.T], axis=1)
    pre = out0 + skip0[...]
    h = jnp.where(pre > 0, pre, jnp.exp(pre) - 1.0)
    H1 = jnp.dot(h, wcat1[...], preferred_element_type=F32)
    h1 = H1[:, :HEADS * OUT_C]
    hd1 = H1[:, HEADS * OUT_C:2 * HEADS * OUT_C]
    for c in range((HEADS * OUT_C) // (CPT * 16)):
        h1_ref[c] = h1[:, c * CPT * 16:(c + 1) * CPT * 16]
    skip1_ref[...] = H1[:, 2 * HEADS * OUT_C:] + bsk1[...]
    as0 = jnp.sum(h1[:, :OUT_C] * a1s[0:1, :], axis=1)
    as1 = jnp.sum(h1[:, OUT_C:] * a1s[1:2, :], axis=1)
    ad0 = jnp.sum(hd1[:, :OUT_C] * a1d[0:1, :], axis=1)
    ad1 = jnp.sum(hd1[:, OUT_C:] * a1d[1:2, :], axis=1)
    tsrc_ref[0:1, :] = as0[None, :]
    tsrc_ref[1:2, :] = as1[None, :]
    m0 = jnp.max(jnp.abs(as0))
    m1 = jnp.max(jnp.abs(as1))
    tdst_ref[0:1, :] = ad0[None, :]
    tdst_ref[1:2, :] = ad1[None, :]
    tdst_ref[2:3, :] = jnp.abs(ad0[None, :]) + m0
    tdst_ref[3:4, :] = jnp.abs(ad1[None, :]) + m1


def _tc_post_body(acc, den, skip1, o_ref):
    d = jnp.sum(den[...], axis=0)
    inv0 = 1.0 / (d[0:1, :1000] + 1e-16)
    inv1 = 1.0 / (d[1:2, :1000] + 1e-16)
    accs = jnp.sum(acc[:, :1000, :], axis=0)
    m0 = accs[:, :OUT_C] * inv0.T
    m1 = accs[:, OUT_C:] * inv1.T
    o = 0.5 * (m0 + m1) + skip1[...]
    t = o - jnp.max(o, axis=1, keepdims=True)
    o_ref[...] = t - jnp.log(jnp.sum(jnp.exp(t), axis=1, keepdims=True))


def kernel(x, edge_index0, edge_index1, n_target0, n_target1,
           W0_src, W0_dst, a0_src, a0_dst, Wskip0, bskip0,
           W1_src, W1_dst, a1_src, a1_dst, Wskip1, bskip1):
    x5 = x[:5000]
    wcat0 = jnp.concatenate([W0_src, W0_dst, Wskip0], axis=1)
    wcat1 = jnp.concatenate([W1_src, W1_dst, Wskip1], axis=1)

    ht0, tsrc0, tdst0, skip0 = pl.pallas_call(
        _tc_pre_body,
        out_shape=[jax.ShapeDtypeStruct(
                       ((HEADS * HIDDEN) // (CPT * 16), 5000, CPT * 16), F32),
                   jax.ShapeDtypeStruct((2, 5000), F32),
                   jax.ShapeDtypeStruct((4, 1000), F32),
                   jax.ShapeDtypeStruct((1000, HEADS * HIDDEN), F32)],
    )(x5, wcat0, a0_src, a0_dst, bskip0.reshape(1, -1))

    acc0, den0 = _edge0(edge_index0[0], edge_index0[1], ht0, tsrc0, tdst0)

    ht1, tsrc1, tdst1, skip1 = pl.pallas_call(
        _tc_mid_body,
        out_shape=[jax.ShapeDtypeStruct(
                       ((HEADS * OUT_C) // (CPT * 16), 1000, CPT * 16), F32),
                   jax.ShapeDtypeStruct((2, 1000), F32),
                   jax.ShapeDtypeStruct((4, 1000), F32),
                   jax.ShapeDtypeStruct((1000, OUT_C), F32)],
    )(acc0, den0, skip0, wcat1, a1_src, a1_dst, bskip1.reshape(1, -1))

    acc1, den1 = _edge1(edge_index1[0], edge_index1[1], ht1, tsrc1, tdst1)

    return pl.pallas_call(
        _tc_post_body,
        out_shape=jax.ShapeDtypeStruct((1000, OUT_C), F32),
    )(acc1, den1, skip1)
